# Initial kernel scaffold; baseline (speedup 1.0000x reference)
import jax
import jax.numpy as jnp
from jax.experimental import pallas as pl


def kernel(nodes, edges, senders, receivers, W_msg, b_msg, W_node, b_node):
    raise NotImplementedError("write your pallas kernel here")


# speedup vs baseline: 3.2725x; 3.2725x over previous
import functools

import jax
import jax.numpy as jnp
from jax import lax
from jax.experimental import pallas as pl
from jax.experimental.pallas import tpu as pltpu
from jax.experimental.pallas import tpu_sc as plsc

N = 10000
D = 128
E = 320000
NC = 2
NS = 16
NW = NC * NS
EPW = E // NW
BATCH = 80
NB = EPW // BATCH
CH = 624
TAIL = N - NS * CH

_HIGH = lax.Precision.HIGHEST


def _dot_t(a, w):
    return lax.dot_general(a, w, (((1,), (1,)), ((), ())),
                           precision=_HIGH, preferred_element_type=jnp.float32)


def _pre_body(nodes_ref, wm_ref, bm_ref, m_ref):
    m_ref[...] = _dot_t(nodes_ref[...], wm_ref[...]) + bm_ref[...]


def _edges_body(e_ref, o_ref):
    o_ref[...] = e_ref[...] + e_ref[...]


def _post_body(nodes_ref, acc_ref, wn_ref, bn_ref, out_ref):
    nodes = nodes_ref[...]
    magg = acc_ref[0] + acc_ref[1]
    w = wn_ref[...]
    out_ref[...] = (nodes + _dot_t(nodes, w[:, :D]) + _dot_t(magg, w[:, D:])
                    + bn_ref[...])


@functools.partial(
    pl.kernel,
    out_type=jax.ShapeDtypeStruct((NC, N, D), jnp.float32),
    mesh=plsc.VectorSubcoreMesh(core_axis_name="c", subcore_axis_name="s",
                                num_cores=NC, num_subcores=NS),
    scratch_types=[
        pltpu.VMEM((NB, BATCH), jnp.int32),
        pltpu.VMEM((NB, BATCH), jnp.int32),
        pltpu.VMEM((BATCH, D), jnp.float32),
        pltpu.VMEM_SHARED((N, D), jnp.float32),
    ],
)
def _sc_segsum(m_hbm, s_hbm, r_hbm, z_hbm, out_hbm, s_idx, r_idx, rows, acc):
    c = lax.axis_index("c")
    s = lax.axis_index("s")
    wid = c * NS + s
    pltpu.sync_copy(s_hbm.at[wid], s_idx)
    pltpu.sync_copy(r_hbm.at[wid], r_idx)
    base = s * CH
    pltpu.sync_copy(z_hbm, acc.at[pl.ds(base, CH)])

    @pl.when(s == 0)
    def _():
        pltpu.sync_copy(z_hbm.at[pl.ds(0, TAIL)], acc.at[pl.ds(NS * CH, TAIL)])

    plsc.subcore_barrier()

    @pl.loop(0, NB)
    def _(i):
        pltpu.sync_copy(m_hbm.at[s_idx.at[i]], rows)
        pltpu.sync_copy(rows, acc.at[r_idx.at[i]], add=True)

    plsc.subcore_barrier()
    pltpu.sync_copy(acc.at[pl.ds(base, CH)], out_hbm.at[c, pl.ds(base, CH)])

    @pl.when(s == 0)
    def _():
        pltpu.sync_copy(acc.at[pl.ds(NS * CH, TAIL)],
                        out_hbm.at[c, pl.ds(NS * CH, TAIL)])


def kernel(nodes, edges, senders, receivers, W_msg, b_msg, W_node, b_node):
    senders = senders.astype(jnp.int32).reshape(NW, NB, BATCH)
    receivers = receivers.astype(jnp.int32).reshape(NW, NB, BATCH)
    zeros = jnp.zeros((CH, D), jnp.float32)

    m = pl.pallas_call(
        _pre_body,
        out_shape=jax.ShapeDtypeStruct((N, D), jnp.float32),
    )(nodes, W_msg, b_msg.reshape(1, D))

    e2 = edges.reshape(E // 32, 128)
    new_edges = pl.pallas_call(
        _edges_body,
        out_shape=jax.ShapeDtypeStruct(e2.shape, e2.dtype),
    )(e2).reshape(edges.shape)

    acc = _sc_segsum(m, senders, receivers, zeros)

    new_nodes = pl.pallas_call(
        _post_body,
        out_shape=jax.ShapeDtypeStruct((N, D), jnp.float32),
        compiler_params=pltpu.CompilerParams(vmem_limit_bytes=64 * 2**20),
    )(nodes, acc, W_node, b_node.reshape(1, D))

    return (new_nodes, new_edges)


# speedup vs baseline: 4.8025x; 1.4675x over previous
import functools

import jax
import jax.numpy as jnp
from jax import lax
from jax.experimental import pallas as pl
from jax.experimental.pallas import tpu as pltpu
from jax.experimental.pallas import tpu_sc as plsc

N = 10000
D = 128
E = 320000
NC = 2
NS = 16
NW = NC * NS
EPW = E // NW
BATCH = 80
NB = EPW // BATCH
CH = 624
TAIL = N - NS * CH

_HIGH = lax.Precision.HIGHEST


def _dot_t(a, w):
    return lax.dot_general(a, w, (((1,), (1,)), ((), ())),
                           precision=_HIGH, preferred_element_type=jnp.float32)


def _pre_body(nodes_ref, wm_ref, bm_ref, m_ref):
    m_ref[...] = _dot_t(nodes_ref[...], wm_ref[...]) + bm_ref[...]


def _edges_body(e_ref, o_ref):
    o_ref[...] = e_ref[...] + e_ref[...]


def _post_body(nodes_ref, acc_ref, wn_ref, bn_ref, out_ref):
    nodes = nodes_ref[...]
    magg = acc_ref[0] + acc_ref[1]
    w = wn_ref[...]
    out_ref[...] = (nodes + _dot_t(nodes, w[:, :D]) + _dot_t(magg, w[:, D:])
                    + bn_ref[...])


@functools.partial(
    pl.kernel,
    out_type=jax.ShapeDtypeStruct((NC, N, D), jnp.float32),
    mesh=plsc.VectorSubcoreMesh(core_axis_name="c", subcore_axis_name="s",
                                num_cores=NC, num_subcores=NS),
    scratch_types=[
        pltpu.VMEM((NB, BATCH), jnp.int32),
        pltpu.VMEM((NB, BATCH), jnp.int32),
        pltpu.VMEM((BATCH, D), jnp.float32),
        pltpu.VMEM_SHARED((N, D), jnp.float32),
    ],
)
def _sc_segsum(m_hbm, s_hbm, r_hbm, z_hbm, out_hbm, s_idx, r_idx, rows, acc):
    c = lax.axis_index("c")
    s = lax.axis_index("s")
    wid = c * NS + s
    pltpu.sync_copy(s_hbm.at[wid], s_idx)
    pltpu.sync_copy(r_hbm.at[wid], r_idx)
    base = s * CH
    pltpu.sync_copy(z_hbm, acc.at[pl.ds(base, CH)])

    @pl.when(s == 0)
    def _():
        pltpu.sync_copy(z_hbm.at[pl.ds(0, TAIL)], acc.at[pl.ds(NS * CH, TAIL)])

    plsc.subcore_barrier()

    @pl.loop(0, NB)
    def _(i):
        pltpu.sync_copy(m_hbm.at[s_idx.at[i]], rows)
        pltpu.sync_copy(rows, acc.at[r_idx.at[i]], add=True)

    plsc.subcore_barrier()
    pltpu.sync_copy(acc.at[pl.ds(base, CH)], out_hbm.at[c, pl.ds(base, CH)])

    @pl.when(s == 0)
    def _():
        pltpu.sync_copy(acc.at[pl.ds(NS * CH, TAIL)],
                        out_hbm.at[c, pl.ds(NS * CH, TAIL)])


def kernel(nodes, edges, senders, receivers, W_msg, b_msg, W_node, b_node):
    senders = senders.astype(jnp.int32).reshape(NW, NB, BATCH)
    receivers = receivers.astype(jnp.int32).reshape(NW, NB, BATCH)
    zeros = jnp.zeros((CH, D), jnp.float32)

    m = pl.pallas_call(
        _pre_body,
        out_shape=jax.ShapeDtypeStruct((N, D), jnp.float32),
    )(nodes, W_msg, b_msg.reshape(1, D))

    eblk = 8000
    new_edges = pl.pallas_call(
        _edges_body,
        grid=(E // eblk,),
        in_specs=[pl.BlockSpec((eblk, 4), lambda i: (i, 0))],
        out_specs=pl.BlockSpec((eblk, 4), lambda i: (i, 0)),
        out_shape=jax.ShapeDtypeStruct(edges.shape, edges.dtype),
    )(edges)

    acc = _sc_segsum(m, senders, receivers, zeros)

    new_nodes = pl.pallas_call(
        _post_body,
        out_shape=jax.ShapeDtypeStruct((N, D), jnp.float32),
        compiler_params=pltpu.CompilerParams(vmem_limit_bytes=64 * 2**20),
    )(nodes, acc, W_node, b_node.reshape(1, D))

    return (new_nodes, new_edges)


# speedup vs baseline: 8.3130x; 1.7310x over previous
import functools

import jax
import jax.numpy as jnp
from jax import lax
from jax.experimental import pallas as pl
from jax.experimental.pallas import tpu as pltpu
from jax.experimental.pallas import tpu_sc as plsc

N = 10000
D = 128
E = 320000
NC = 2
NS = 16
NW = NC * NS
EPW = E // NW
BATCH = 80
NB = EPW // BATCH
K = 5
GROUPS = NB // K
CH = 624
TAIL = N - NS * CH

_HIGH = lax.Precision.HIGHEST


def _dot_t(a, w):
    return lax.dot_general(a, w, (((1,), (1,)), ((), ())),
                           precision=_HIGH, preferred_element_type=jnp.float32)


def _pre_body(nodes_ref, wm_ref, bm_ref, m_ref):
    m_ref[...] = _dot_t(nodes_ref[...], wm_ref[...]) + bm_ref[...]


def _edges_body(e_ref, o_ref):
    o_ref[...] = e_ref[...] + e_ref[...]


def _post_body(nodes_ref, acc_ref, wn_ref, bn_ref, out_ref):
    nodes = nodes_ref[...]
    magg = acc_ref[0] + acc_ref[1]
    w = wn_ref[...]
    out_ref[...] = (nodes + _dot_t(nodes, w[:, :D]) + _dot_t(magg, w[:, D:])
                    + bn_ref[...])


@functools.partial(
    pl.kernel,
    out_type=jax.ShapeDtypeStruct((NC, N, D), jnp.float32),
    mesh=plsc.VectorSubcoreMesh(core_axis_name="c", subcore_axis_name="s",
                                num_cores=NC, num_subcores=NS),
    scratch_types=[
        pltpu.VMEM((NB, BATCH), jnp.int32),
        pltpu.VMEM((BATCH,), jnp.int32),
        pltpu.VMEM((BATCH,), jnp.int32),
        pltpu.VMEM((BATCH, D), jnp.float32),
        pltpu.VMEM((BATCH, D), jnp.float32),
        pltpu.VMEM_SHARED((N, D), jnp.float32),
        pltpu.SemaphoreType.DMA,
        pltpu.SemaphoreType.DMA,
        pltpu.SemaphoreType.DMA,
        pltpu.SemaphoreType.DMA,
    ],
)
def _sc_segsum(m_hbm, s_hbm, r_hbm, z_hbm, out_hbm, s_idx, riA, riB,
               rowA, rowB, acc, gsA, gsB, ssA, ssB):
    c = lax.axis_index("c")
    s = lax.axis_index("s")
    wid = c * NS + s
    pltpu.sync_copy(s_hbm.at[wid], s_idx)
    base = s * CH
    pltpu.sync_copy(z_hbm, acc.at[pl.ds(base, CH)])

    @pl.when(s == 0)
    def _():
        pltpu.sync_copy(z_hbm.at[pl.ds(0, TAIL)], acc.at[pl.ds(NS * CH, TAIL)])

    plsc.subcore_barrier()

    def fire_g(buf, ri, gsem, i):
        pltpu.async_copy(r_hbm.at[wid, i], ri, gsem)
        pltpu.async_copy(m_hbm.at[s_idx.at[i]], buf, gsem)

    def fire_s(buf, ri, ssem, i):
        pltpu.async_copy(buf, acc.at[ri], ssem, add=True)

    def drain_g(buf, ri, gsem):
        pltpu.make_async_copy(m_hbm.at[pl.ds(0, BATCH)], buf, gsem).wait()
        pltpu.make_async_copy(r_hbm.at[0, 0], ri, gsem).wait()

    def drain_s(buf, ssem):
        pltpu.make_async_copy(m_hbm.at[pl.ds(0, BATCH)], buf, ssem).wait()

    fire_g(rowA, riA, gsA, 0)

    @pl.loop(0, (NB - 1) // 2)
    def _(j):
        i0 = 2 * j
        drain_g(rowA, riA, gsA)

        @pl.when(j > 0)
        def _():
            drain_s(rowB, ssB)

        fire_g(rowB, riB, gsB, i0 + 1)
        fire_s(rowA, riA, ssA, i0)
        drain_g(rowB, riB, gsB)
        drain_s(rowA, ssA)
        fire_g(rowA, riA, gsA, i0 + 2)
        fire_s(rowB, riB, ssB, i0 + 1)

    drain_g(rowA, riA, gsA)
    drain_s(rowB, ssB)
    fire_s(rowA, riA, ssA, NB - 1)
    drain_s(rowA, ssA)

    plsc.subcore_barrier()
    pltpu.sync_copy(acc.at[pl.ds(base, CH)], out_hbm.at[c, pl.ds(base, CH)])

    @pl.when(s == 0)
    def _():
        pltpu.sync_copy(acc.at[pl.ds(NS * CH, TAIL)],
                        out_hbm.at[c, pl.ds(NS * CH, TAIL)])


def kernel(nodes, edges, senders, receivers, W_msg, b_msg, W_node, b_node):
    senders = senders.astype(jnp.int32).reshape(NW, NB, BATCH)
    receivers = receivers.astype(jnp.int32).reshape(NW, NB, BATCH)
    zeros = jnp.zeros((CH, D), jnp.float32)

    m = pl.pallas_call(
        _pre_body,
        out_shape=jax.ShapeDtypeStruct((N, D), jnp.float32),
    )(nodes, W_msg, b_msg.reshape(1, D))

    new_edges = edges + edges

    acc = _sc_segsum(m, senders, receivers, zeros)

    new_nodes = pl.pallas_call(
        _post_body,
        out_shape=jax.ShapeDtypeStruct((N, D), jnp.float32),
        compiler_params=pltpu.CompilerParams(vmem_limit_bytes=64 * 2**20),
    )(nodes, acc, W_node, b_node.reshape(1, D))

    return (new_nodes, new_edges)
